# baseline (speedup 1.0000x reference)
import functools

import jax
import jax.numpy as jnp
from jax.experimental import pallas as pl
from jax.experimental.pallas import tpu as pltpu

N = 10000
E = 320000
B = 256
D = 128
H = 2 * D


def _bn(h):
    m = jnp.mean(h, axis=0, keepdims=True)
    v = jnp.mean((h - m) ** 2, axis=0, keepdims=True)
    return (h - m) / jnp.sqrt(v + 1e-5)


def _dot(a, b):
    return jnp.dot(a, b, preferred_element_type=jnp.float32)


def _enc_body(x_ref, w_ref, b_ref, o_ref):
    o_ref[...] = _dot(x_ref[...], w_ref[...]) + b_ref[...]


def _encode(x, enc_W, enc_b):
    return pl.pallas_call(
        _enc_body,
        out_shape=jax.ShapeDtypeStruct((N, D), jnp.float32),
    )(x, enc_W, enc_b.reshape(1, D))


def _gin_body(hv_ref, agg_ref, w1_ref, b1_ref, w2_ref, b2_ref, o_ref, *, relu_last):
    hv = hv_ref[...]
    m = hv + agg_ref[...]
    t = jax.nn.relu(_bn(_dot(m, w1_ref[...]) + b1_ref[...]))
    u = _bn(_dot(t, w2_ref[...]) + b2_ref[...])
    if relu_last:
        u = jax.nn.relu(u)
    o_ref[...] = u + hv


def _gin_dense(hv, agg, W1, b1, W2, b2, relu_last):
    return pl.pallas_call(
        functools.partial(_gin_body, relu_last=relu_last),
        out_shape=jax.ShapeDtypeStruct((N, D), jnp.float32),
    )(hv, agg, W1, b1.reshape(1, H), W2, b2.reshape(1, D))


def _vn_body(vs_ref, vn_ref, w1_ref, b1_ref, w2_ref, b2_ref, o_ref):
    vt = vs_ref[...] + vn_ref[...]
    vt = jax.nn.relu(_bn(_dot(vt, w1_ref[...]) + b1_ref[...]))
    o_ref[...] = jax.nn.relu(_dot(vt, w2_ref[...]) + b2_ref[...])


def _vn_update(vsum, vn, W1, b1, W2, b2):
    return pl.pallas_call(
        _vn_body,
        out_shape=jax.ShapeDtypeStruct((B, D), jnp.float32),
    )(vsum, vn, W1, b1.reshape(1, H), W2, b2.reshape(1, D))


def _head_body(g_ref, mol_ref, m0w, m0b, m1w, m1b, m2w, m2b,
               l0w, l0b, l1w, l1b, l2w, l2b, o_ref):
    y = jax.nn.relu(_bn(_dot(mol_ref[...], m0w[...]) + m0b[...]))
    y = jax.nn.relu(_bn(_dot(y, m1w[...]) + m1b[...]))
    y = _dot(y, m2w[...]) + m2b[...]
    z = jnp.concatenate([g_ref[...], y], axis=1)
    z = jax.nn.relu(_bn(_dot(z, l0w[...]) + l0b[...]))
    z = jax.nn.relu(_bn(_dot(z, l1w[...]) + l1b[...]))
    o_ref[...] = _dot(z, l2w[...]) + l2b[...]


def _head(g, mol_attr, mlp0_W, mlp0_b, mlp1_W, mlp1_b, mlp2_W, mlp2_b,
          last0_W, last0_b, last1_W, last1_b, last2_W, last2_b):
    return pl.pallas_call(
        _head_body,
        out_shape=jax.ShapeDtypeStruct((B, 1), jnp.float32),
    )(g, mol_attr,
      mlp0_W, mlp0_b.reshape(1, D), mlp1_W, mlp1_b.reshape(1, D),
      mlp2_W, mlp2_b.reshape(1, D),
      last0_W, last0_b.reshape(1, D), last1_W, last1_b.reshape(1, D),
      last2_W, last2_b.reshape(1, 1))


def _edge_agg(h, src, dst):
    return jax.ops.segment_sum(h[src], dst, num_segments=N)


def _batch_sum(h, batch):
    return jax.ops.segment_sum(h, batch, num_segments=B)


def _vn_gather_add(h, vn, batch):
    return h + vn[batch]


def kernel(x, edge_index, batch, mol_attr, enc_W, enc_b,
           gin0_W1, gin0_b1, gin0_W2, gin0_b2,
           gin1_W1, gin1_b1, gin1_W2, gin1_b2,
           gin2_W1, gin2_b1, gin2_W2, gin2_b2,
           vn0_W1, vn0_b1, vn0_W2, vn0_b2,
           vn1_W1, vn1_b1, vn1_W2, vn1_b2,
           mlp0_W, mlp0_b, mlp1_W, mlp1_b, mlp2_W, mlp2_b,
           last0_W, last0_b, last1_W, last1_b, last2_W, last2_b):
    gin = [(gin0_W1, gin0_b1, gin0_W2, gin0_b2),
           (gin1_W1, gin1_b1, gin1_W2, gin1_b2),
           (gin2_W1, gin2_b1, gin2_W2, gin2_b2)]
    vnp = [(vn0_W1, vn0_b1, vn0_W2, vn0_b2),
           (vn1_W1, vn1_b1, vn1_W2, vn1_b2)]
    src = edge_index[0]
    dst = edge_index[1]

    h = _encode(x, enc_W, enc_b)
    vn = None
    for l in range(3):
        hv = h if vn is None else _vn_gather_add(h, vn, batch)
        agg = _edge_agg(hv, src, dst)
        W1, b1, W2, b2 = gin[l]
        h = _gin_dense(hv, agg, W1, b1, W2, b2, relu_last=(l < 2))
        if l < 2:
            vsum = _batch_sum(h, batch)
            if vn is None:
                vn = jnp.zeros((B, D), jnp.float32)
            vn = _vn_update(vsum, vn, *vnp[l])
    g = _batch_sum(h, batch)
    return _head(g, mol_attr, mlp0_W, mlp0_b, mlp1_W, mlp1_b, mlp2_W, mlp2_b,
                 last0_W, last0_b, last1_W, last1_b, last2_W, last2_b)


# speedup vs baseline: 4.3913x; 4.3913x over previous
import functools

import jax
import jax.numpy as jnp
from jax import lax
from jax.experimental import pallas as pl
from jax.experimental.pallas import tpu as pltpu
from jax.experimental.pallas import tpu_sc as plsc

N = 10000
E = 320000
B = 256
D = 128
H = 2 * D

_NC = 2
_NS = 16
_NW = _NC * _NS
_EPW = E // _NW
_K = 80
_NCH = _EPW // _K
_RPT = 624
_TAIL = N - _NS * _RPT


def _bn(h):
    m = jnp.mean(h, axis=0, keepdims=True)
    v = jnp.mean((h - m) ** 2, axis=0, keepdims=True)
    return (h - m) / jnp.sqrt(v + 1e-5)


def _dot(a, b):
    return jnp.dot(a, b, preferred_element_type=jnp.float32)


def _onehot(batch_col, rows):
    ids = lax.broadcasted_iota(jnp.int32, (rows, B), 1)
    return (batch_col == ids).astype(jnp.float32)


def _edge_agg_body(hv_hbm, src_hbm, dst_hbm, out_hbm, idx_s, idx_d, rows, acc, sem):
    c = lax.axis_index("c")
    s = lax.axis_index("s")
    wid = c * _NS + s
    r0 = s * _RPT
    pltpu.sync_copy(hv_hbm.at[pl.ds(r0, _RPT)], acc.at[pl.ds(r0, _RPT)])

    @pl.when(s == _NS - 1)
    def _init_tail():
        t0 = _NS * _RPT
        pltpu.sync_copy(hv_hbm.at[pl.ds(t0, _TAIL)], acc.at[pl.ds(t0, _TAIL)])

    plsc.subcore_barrier()
    ebase = wid * _EPW

    def body(j, carry):
        base = ebase + j * _K
        pltpu.sync_copy(src_hbm.at[pl.ds(base, _K)], idx_s)
        pltpu.sync_copy(dst_hbm.at[pl.ds(base, _K)], idx_d)
        pltpu.async_copy(hv_hbm.at[idx_s], rows, sem).wait()
        pltpu.sync_copy(rows, acc.at[idx_d], add=True)
        return carry

    lax.fori_loop(0, _NCH, body, 0)
    plsc.subcore_barrier()
    pltpu.sync_copy(acc.at[pl.ds(r0, _RPT)], out_hbm.at[c, pl.ds(r0, _RPT)])

    @pl.when(s == _NS - 1)
    def _out_tail():
        t0 = _NS * _RPT
        pltpu.sync_copy(acc.at[pl.ds(t0, _TAIL)], out_hbm.at[c, pl.ds(t0, _TAIL)])


def _edge_agg(hv, src, dst):
    mesh = plsc.VectorSubcoreMesh(core_axis_name="c", subcore_axis_name="s")
    return pl.kernel(
        _edge_agg_body,
        out_type=jax.ShapeDtypeStruct((_NC, N, D), jnp.float32),
        mesh=mesh,
        scratch_types=[
            pltpu.VMEM((_K,), jnp.int32),
            pltpu.VMEM((_K,), jnp.int32),
            pltpu.VMEM((_K, D), jnp.float32),
            pltpu.VMEM_SHARED((N, D), jnp.float32),
            pltpu.SemaphoreType.DMA,
        ],
    )(hv, src, dst)


def _enc_body(x_ref, w_ref, b_ref, o_ref):
    o_ref[...] = _dot(x_ref[...], w_ref[...]) + b_ref[...]


def _encode(x, enc_W, enc_b):
    return pl.pallas_call(
        _enc_body,
        out_shape=jax.ShapeDtypeStruct((N, D), jnp.float32),
    )(x, enc_W, enc_b.reshape(1, D))


def _gin_body(hv_ref, agg_ref, batch_ref, w1_ref, b1_ref, w2_ref, b2_ref,
              o_ref, vs_ref, *, relu_last):
    hv = hv_ref[...]
    m = agg_ref[0] + agg_ref[1] - hv
    t = jax.nn.relu(_bn(_dot(m, w1_ref[...]) + b1_ref[...]))
    u = _bn(_dot(t, w2_ref[...]) + b2_ref[...])
    if relu_last:
        u = jax.nn.relu(u)
    hn = u + hv
    o_ref[...] = hn
    onehot = _onehot(batch_ref[...], N)
    vs_ref[...] = lax.dot_general(
        onehot, hn, (((0,), (0,)), ((), ())),
        precision=lax.Precision.HIGHEST, preferred_element_type=jnp.float32)


def _gin_dense(hv, agg2, batch_col, W1, b1, W2, b2, relu_last):
    return pl.pallas_call(
        functools.partial(_gin_body, relu_last=relu_last),
        out_shape=(jax.ShapeDtypeStruct((N, D), jnp.float32),
                   jax.ShapeDtypeStruct((B, D), jnp.float32)),
    )(hv, agg2, batch_col, W1, b1.reshape(1, H), W2, b2.reshape(1, D))


def _vn_body(vs_ref, vn_ref, h_ref, batch_ref, w1_ref, b1_ref, w2_ref, b2_ref,
             vno_ref, hv_ref):
    vt = vs_ref[...] + vn_ref[...]
    vt = jax.nn.relu(_bn(_dot(vt, w1_ref[...]) + b1_ref[...]))
    vn = jax.nn.relu(_dot(vt, w2_ref[...]) + b2_ref[...])
    vno_ref[...] = vn
    onehot = _onehot(batch_ref[...], N)
    hv_ref[...] = h_ref[...] + _dot(onehot, vn)


def _vn_update(vsum, vn, h, batch_col, W1, b1, W2, b2):
    return pl.pallas_call(
        _vn_body,
        out_shape=(jax.ShapeDtypeStruct((B, D), jnp.float32),
                   jax.ShapeDtypeStruct((N, D), jnp.float32)),
    )(vsum, vn, h, batch_col, W1, b1.reshape(1, H), W2, b2.reshape(1, D))


def _head_body(g_ref, mol_ref, m0w, m0b, m1w, m1b, m2w, m2b,
               l0w, l0b, l1w, l1b, l2w, l2b, o_ref):
    y = jax.nn.relu(_bn(_dot(mol_ref[...], m0w[...]) + m0b[...]))
    y = jax.nn.relu(_bn(_dot(y, m1w[...]) + m1b[...]))
    y = _dot(y, m2w[...]) + m2b[...]
    z = jnp.concatenate([g_ref[...], y], axis=1)
    z = jax.nn.relu(_bn(_dot(z, l0w[...]) + l0b[...]))
    z = jax.nn.relu(_bn(_dot(z, l1w[...]) + l1b[...]))
    o_ref[...] = _dot(z, l2w[...]) + l2b[...]


def _head(g, mol_attr, mlp0_W, mlp0_b, mlp1_W, mlp1_b, mlp2_W, mlp2_b,
          last0_W, last0_b, last1_W, last1_b, last2_W, last2_b):
    return pl.pallas_call(
        _head_body,
        out_shape=jax.ShapeDtypeStruct((B, 1), jnp.float32),
    )(g, mol_attr,
      mlp0_W, mlp0_b.reshape(1, D), mlp1_W, mlp1_b.reshape(1, D),
      mlp2_W, mlp2_b.reshape(1, D),
      last0_W, last0_b.reshape(1, D), last1_W, last1_b.reshape(1, D),
      last2_W, last2_b.reshape(1, 1))


def kernel(x, edge_index, batch, mol_attr, enc_W, enc_b,
           gin0_W1, gin0_b1, gin0_W2, gin0_b2,
           gin1_W1, gin1_b1, gin1_W2, gin1_b2,
           gin2_W1, gin2_b1, gin2_W2, gin2_b2,
           vn0_W1, vn0_b1, vn0_W2, vn0_b2,
           vn1_W1, vn1_b1, vn1_W2, vn1_b2,
           mlp0_W, mlp0_b, mlp1_W, mlp1_b, mlp2_W, mlp2_b,
           last0_W, last0_b, last1_W, last1_b, last2_W, last2_b):
    gin = [(gin0_W1, gin0_b1, gin0_W2, gin0_b2),
           (gin1_W1, gin1_b1, gin1_W2, gin1_b2),
           (gin2_W1, gin2_b1, gin2_W2, gin2_b2)]
    vnp = [(vn0_W1, vn0_b1, vn0_W2, vn0_b2),
           (vn1_W1, vn1_b1, vn1_W2, vn1_b2)]
    src = edge_index[0]
    dst = edge_index[1]
    batch_col = batch.astype(jnp.int32).reshape(N, 1)

    hv = _encode(x, enc_W, enc_b)
    h = hv
    vn = None
    vsum = None
    for l in range(3):
        agg2 = _edge_agg(hv, src, dst)
        W1, b1, W2, b2 = gin[l]
        h, vsum = _gin_dense(hv, agg2, batch_col, W1, b1, W2, b2,
                             relu_last=(l < 2))
        if l < 2:
            if vn is None:
                vn = jnp.zeros((B, D), jnp.float32)
            vn, hv = _vn_update(vsum, vn, h, batch_col, *vnp[l])
    return _head(vsum, mol_attr,
                 mlp0_W, mlp0_b, mlp1_W, mlp1_b, mlp2_W, mlp2_b,
                 last0_W, last0_b, last1_W, last1_b, last2_W, last2_b)


# speedup vs baseline: 9.2515x; 2.1068x over previous
import functools

import jax
import jax.numpy as jnp
from jax import lax
from jax.experimental import pallas as pl
from jax.experimental.pallas import tpu as pltpu
from jax.experimental.pallas import tpu_sc as plsc

N = 10000
E = 320000
B = 256
D = 128
H = 2 * D

_NC = 2
_NS = 16
_NW = _NC * _NS
_KE = 80
_NCH = 125
_EPW = _KE * _NCH
_MAINW = _KE * _NCH
_TAILE = (E - _NW * _MAINW) // _NW
_RPT = 624
_TAIL = N - _NS * _RPT


def _bn(h):
    m = jnp.mean(h, axis=0, keepdims=True)
    v = jnp.mean((h - m) ** 2, axis=0, keepdims=True)
    return (h - m) / jnp.sqrt(v + 1e-5)


def _dot(a, b):
    return jnp.dot(a, b, preferred_element_type=jnp.float32)


def _onehot(batch_col, rows):
    ids = lax.broadcasted_iota(jnp.int32, (rows, B), 1)
    return (batch_col == ids).astype(jnp.float32)


_NB = 4
_NMAIN = _NCH - 1


def _edge_agg_body(hv_hbm, src_hbm, dst_hbm, out_hbm, *refs):
    isl = refs[0:4]
    idl = refs[4:8]
    rowsl = refs[8:12]
    acc = refs[12]
    iseml = refs[13:17]
    gseml = refs[17:21]
    c = lax.axis_index("c")
    s = lax.axis_index("s")
    wid = c * _NS + s
    r0 = s * _RPT
    pltpu.sync_copy(hv_hbm.at[pl.ds(r0, _RPT)], acc.at[pl.ds(r0, _RPT)])

    @pl.when(s == _NS - 1)
    def _init_tail():
        t0 = _NS * _RPT
        pltpu.sync_copy(hv_hbm.at[pl.ds(t0, _TAIL)], acc.at[pl.ds(t0, _TAIL)])

    eb = wid * _EPW

    def idx_start(j, b):
        base = eb + j * _KE
        pltpu.async_copy(src_hbm.at[pl.ds(base, _KE)], isl[b], iseml[b])
        pltpu.async_copy(dst_hbm.at[pl.ds(base, _KE)], idl[b], iseml[b])

    def idx_wait(j, b):
        base = eb + j * _KE
        pltpu.make_async_copy(src_hbm.at[pl.ds(base, _KE)], isl[b], iseml[b]).wait()
        pltpu.make_async_copy(dst_hbm.at[pl.ds(base, _KE)], idl[b], iseml[b]).wait()

    plsc.subcore_barrier()

    for b in range(_NB):
        idx_start(b, b)
    for b in range(2):
        idx_wait(b, b)
        pltpu.async_copy(hv_hbm.at[isl[b]], rowsl[b], gseml[b])

    def body(g, carry):
        for b in range(_NB):
            j = _NB * g + b
            pltpu.make_async_copy(hv_hbm.at[isl[b]], rowsl[b], gseml[b]).wait()
            pltpu.sync_copy(rowsl[b], acc.at[idl[b]], add=True)

            @pl.when(j + _NB < _NMAIN)
            def _():
                idx_start(j + _NB, b)

            bg = (b + 2) % _NB

            @pl.when(j + 2 < _NMAIN)
            def _():
                idx_wait(j + 2, bg)
                pltpu.async_copy(hv_hbm.at[isl[bg]], rowsl[bg], gseml[bg])

        return carry

    lax.fori_loop(0, _NMAIN // _NB, body, 0)
    idx_start(_NCH - 1, 0)
    idx_wait(_NCH - 1, 0)
    pltpu.async_copy(hv_hbm.at[isl[0]], rowsl[0], gseml[0]).wait()
    pltpu.sync_copy(rowsl[0], acc.at[idl[0]], add=True)
    plsc.subcore_barrier()
    pltpu.sync_copy(acc.at[pl.ds(r0, _RPT)], out_hbm.at[c, pl.ds(r0, _RPT)])

    @pl.when(s == _NS - 1)
    def _out_tail():
        t0 = _NS * _RPT
        pltpu.sync_copy(acc.at[pl.ds(t0, _TAIL)], out_hbm.at[c, pl.ds(t0, _TAIL)])


def _edge_agg(hv, src, dst):
    mesh = plsc.VectorSubcoreMesh(core_axis_name="c", subcore_axis_name="s")
    return pl.kernel(
        _edge_agg_body,
        out_type=jax.ShapeDtypeStruct((_NC, N, D), jnp.float32),
        mesh=mesh,
        scratch_types=(
            [pltpu.VMEM((_KE,), jnp.int32) for _ in range(2 * _NB)]
            + [pltpu.VMEM((_KE, D), jnp.float32) for _ in range(_NB)]
            + [pltpu.VMEM_SHARED((N, D), jnp.float32)]
            + [pltpu.SemaphoreType.DMA for _ in range(2 * _NB)]
        ),
    )(hv, src, dst)


def _enc_body(x_ref, w_ref, b_ref, o_ref):
    o_ref[...] = _dot(x_ref[...], w_ref[...]) + b_ref[...]


def _encode(x, enc_W, enc_b):
    return pl.pallas_call(
        _enc_body,
        out_shape=jax.ShapeDtypeStruct((N, D), jnp.float32),
    )(x, enc_W, enc_b.reshape(1, D))


def _gin_body(hv_ref, agg_ref, batch_ref, w1_ref, b1_ref, w2_ref, b2_ref,
              o_ref, vs_ref, *, relu_last):
    hv = hv_ref[...]
    m = agg_ref[0] + agg_ref[1] - hv
    t = jax.nn.relu(_bn(_dot(m, w1_ref[...]) + b1_ref[...]))
    u = _bn(_dot(t, w2_ref[...]) + b2_ref[...])
    if relu_last:
        u = jax.nn.relu(u)
    hn = u + hv
    o_ref[...] = hn
    onehot = _onehot(batch_ref[...], N)
    vs_ref[...] = lax.dot_general(
        onehot, hn, (((0,), (0,)), ((), ())),
        precision=lax.Precision.HIGHEST, preferred_element_type=jnp.float32)


def _gin_dense(hv, agg2, batch_col, W1, b1, W2, b2, relu_last):
    return pl.pallas_call(
        functools.partial(_gin_body, relu_last=relu_last),
        out_shape=(jax.ShapeDtypeStruct((N, D), jnp.float32),
                   jax.ShapeDtypeStruct((B, D), jnp.float32)),
    )(hv, agg2, batch_col, W1, b1.reshape(1, H), W2, b2.reshape(1, D))


def _vn_body(vs_ref, vn_ref, h_ref, batch_ref, w1_ref, b1_ref, w2_ref, b2_ref,
             vno_ref, hv_ref):
    vt = vs_ref[...] + vn_ref[...]
    vt = jax.nn.relu(_bn(_dot(vt, w1_ref[...]) + b1_ref[...]))
    vn = jax.nn.relu(_dot(vt, w2_ref[...]) + b2_ref[...])
    vno_ref[...] = vn
    onehot = _onehot(batch_ref[...], N)
    hv_ref[...] = h_ref[...] + _dot(onehot, vn)


def _vn_update(vsum, vn, h, batch_col, W1, b1, W2, b2):
    return pl.pallas_call(
        _vn_body,
        out_shape=(jax.ShapeDtypeStruct((B, D), jnp.float32),
                   jax.ShapeDtypeStruct((N, D), jnp.float32)),
    )(vsum, vn, h, batch_col, W1, b1.reshape(1, H), W2, b2.reshape(1, D))


def _head_body(g_ref, mol_ref, m0w, m0b, m1w, m1b, m2w, m2b,
               l0w, l0b, l1w, l1b, l2w, l2b, o_ref):
    y = jax.nn.relu(_bn(_dot(mol_ref[...], m0w[...]) + m0b[...]))
    y = jax.nn.relu(_bn(_dot(y, m1w[...]) + m1b[...]))
    y = _dot(y, m2w[...]) + m2b[...]
    z = jnp.concatenate([g_ref[...], y], axis=1)
    z = jax.nn.relu(_bn(_dot(z, l0w[...]) + l0b[...]))
    z = jax.nn.relu(_bn(_dot(z, l1w[...]) + l1b[...]))
    o_ref[...] = _dot(z, l2w[...]) + l2b[...]


def _head(g, mol_attr, mlp0_W, mlp0_b, mlp1_W, mlp1_b, mlp2_W, mlp2_b,
          last0_W, last0_b, last1_W, last1_b, last2_W, last2_b):
    return pl.pallas_call(
        _head_body,
        out_shape=jax.ShapeDtypeStruct((B, 1), jnp.float32),
    )(g, mol_attr,
      mlp0_W, mlp0_b.reshape(1, D), mlp1_W, mlp1_b.reshape(1, D),
      mlp2_W, mlp2_b.reshape(1, D),
      last0_W, last0_b.reshape(1, D), last1_W, last1_b.reshape(1, D),
      last2_W, last2_b.reshape(1, 1))


def kernel(x, edge_index, batch, mol_attr, enc_W, enc_b,
           gin0_W1, gin0_b1, gin0_W2, gin0_b2,
           gin1_W1, gin1_b1, gin1_W2, gin1_b2,
           gin2_W1, gin2_b1, gin2_W2, gin2_b2,
           vn0_W1, vn0_b1, vn0_W2, vn0_b2,
           vn1_W1, vn1_b1, vn1_W2, vn1_b2,
           mlp0_W, mlp0_b, mlp1_W, mlp1_b, mlp2_W, mlp2_b,
           last0_W, last0_b, last1_W, last1_b, last2_W, last2_b):
    gin = [(gin0_W1, gin0_b1, gin0_W2, gin0_b2),
           (gin1_W1, gin1_b1, gin1_W2, gin1_b2),
           (gin2_W1, gin2_b1, gin2_W2, gin2_b2)]
    vnp = [(vn0_W1, vn0_b1, vn0_W2, vn0_b2),
           (vn1_W1, vn1_b1, vn1_W2, vn1_b2)]
    src = edge_index[0]
    dst = edge_index[1]
    batch_col = batch.astype(jnp.int32).reshape(N, 1)

    hv = _encode(x, enc_W, enc_b)
    h = hv
    vn = None
    vsum = None
    for l in range(3):
        agg2 = _edge_agg(hv, src, dst)
        W1, b1, W2, b2 = gin[l]
        h, vsum = _gin_dense(hv, agg2, batch_col, W1, b1, W2, b2,
                             relu_last=(l < 2))
        if l < 2:
            if vn is None:
                vn = jnp.zeros((B, D), jnp.float32)
            vn, hv = _vn_update(vsum, vn, h, batch_col, *vnp[l])
    return _head(vsum, mol_attr,
                 mlp0_W, mlp0_b, mlp1_W, mlp1_b, mlp2_W, mlp2_b,
                 last0_W, last0_b, last1_W, last1_b, last2_W, last2_b)
